# baseline (speedup 1.0000x reference)
import functools

import jax
import jax.numpy as jnp
from jax.experimental import pallas as pl
from jax.experimental.pallas import tpu as pltpu

T = 4096
W = 32

NEG_INF = float("-inf")


def _elu(x):
    return jnp.where(x > 0, x, jnp.exp(x) - 1.0)


def _leaky(x):
    return jnp.where(x > 0, x, 0.01 * x)


def _ln(x):
    d = x.shape[-1]
    ones = jnp.ones((d, 1), jnp.float32)
    mu = _mm(x, ones, 1, 0) * (1.0 / d)
    ex2 = _mm(x * x, ones, 1, 0) * (1.0 / d)
    var = ex2 - mu * mu
    return (x - mu) * jax.lax.rsqrt(var + 1e-5)


def _mm(a, b, ca, cb):
    return jax.lax.dot_general(
        a, b, (((ca,), (cb,)), ((), ())), preferred_element_type=jnp.float32)


def _transfer(x_ref, w1_ref, w2_ref, res_sc):
    res_sc[...] = x_ref[...]
    h = _mm(x_ref[...], w1_ref[...], 1, 1)
    h = _ln(_elu(h))
    h = _mm(h, w2_ref[...], 1, 1)
    return _elu(h + res_sc[...])


def _onehot(ids_row, rows):
    seg_iota = jax.lax.broadcasted_iota(jnp.int32, (rows, T), 0)
    ohb = seg_iota == ids_row
    return ohb, ohb.astype(jnp.float32)


def _body1(lo_ref, wide_ref, ids_ref, x_ref, tgw1_ref, tgw2_ref,
           xm_ref, xg_ref, acc_sc, res_sc, *, nt, b, n):
    i = pl.program_id(0)

    @pl.when(i == 0)
    def _init():
        acc_sc[...] = jnp.zeros_like(acc_sc)

    xg = _transfer(x_ref, tgw1_ref, tgw2_ref, res_sc)
    rmask = jax.lax.broadcasted_iota(jnp.int32, (T, 1), 0) < n - i * T
    xg = jnp.where(rmask, xg, 0.0)
    xg_ref[...] = xg
    ids_row = ids_ref[0]
    lo = lo_ref[i]

    @pl.when(wide_ref[i] == 0)
    def _narrow():
        _, oh = _onehot(ids_row - lo, W)
        sl = pl.ds(lo, W)
        acc_sc[sl, :] += _mm(oh, xg, 1, 0)

    @pl.when(wide_ref[i] != 0)
    def _wide():
        _, oh = _onehot(ids_row, b)
        sl = pl.ds(0, b)
        acc_sc[sl, :] += _mm(oh, xg, 1, 0)

    @pl.when(i == nt - 1)
    def _fin():
        xm_ref[...] = acc_sc[pl.ds(0, b), :]


def _accum(ohb, oh, hlw, m_oldw, d_oldw, s_oldw, xg, a1, att):
    hlg = _mm(oh, hlw, 0, 0)
    v = _leaky(a1 + hlg)
    alpha = _leaky(_mm(att, v, 1, 1))
    amT = jnp.where(ohb, alpha, NEG_INF)
    mt = jnp.max(amT, axis=1, keepdims=True)
    m_new = jnp.maximum(m_oldw, mt)
    scale = jnp.where(m_oldw == NEG_INF, 0.0, jnp.exp(m_oldw - m_new))
    m_fin = jnp.where(m_new == NEG_INF, 0.0, m_new)
    mg = _mm(m_fin, oh, 0, 0)
    w = jnp.exp(alpha - mg)
    ohw = oh * w
    ones_col = jnp.ones((T, 1), jnp.float32)
    s_new = s_oldw * scale + _mm(ohw, xg, 1, 0)
    d_new = d_oldw * scale + _mm(ohw, ones_col, 1, 0)
    return m_new, d_new, s_new


def _body2(lo_ref, wide_ref, ids_ref, xg_ref, xm_ref, xc_ref,
           tlw1_ref, tlw2_ref, w1l_ref, w1r_ref, att_ref, w2_ref,
           bias_ref, exitw_ref, out_ref,
           resc_sc, hl_sc, m_sc, d_sc, s_sc, *, nt, b):
    i = pl.program_id(0)

    @pl.when(i == 0)
    def _init():
        hl_sc[pl.ds(0, b), :] = _mm(jax.nn.relu(xm_ref[...]), w1r_ref[...],
                                    1, 1)
        hl_sc[pl.ds(b, W), :] = jnp.zeros((W, hl_sc.shape[1]), jnp.float32)
        m_sc[...] = jnp.full_like(m_sc, NEG_INF)
        d_sc[...] = jnp.zeros_like(d_sc)
        s_sc[...] = jnp.zeros_like(s_sc)

    xg = xg_ref[...]
    a1 = _mm(xg, w1l_ref[...], 1, 1)
    ids_row = ids_ref[0]
    lo = lo_ref[i]
    att = att_ref[...]

    @pl.when(wide_ref[i] == 0)
    def _narrow():
        ohb, oh = _onehot(ids_row - lo, W)
        sl = pl.ds(lo, W)
        m_new, d_new, s_new = _accum(
            ohb, oh, hl_sc[sl, :], m_sc[sl, :], d_sc[sl, :], s_sc[sl, :],
            xg, a1, att)
        m_sc[sl, :] = m_new
        d_sc[sl, :] = d_new
        s_sc[sl, :] = s_new

    @pl.when(wide_ref[i] != 0)
    def _wide():
        ohb, oh = _onehot(ids_row, b)
        sl = pl.ds(0, b)
        m_new, d_new, s_new = _accum(
            ohb, oh, hl_sc[sl, :], m_sc[sl, :], d_sc[sl, :], s_sc[sl, :],
            xg, a1, att)
        m_sc[sl, :] = m_new
        d_sc[sl, :] = d_new
        s_sc[sl, :] = s_new

    @pl.when(i == nt - 1)
    def _fin():
        xm = jax.nn.relu(xm_ref[...])
        sb = pl.ds(0, b)
        outb = _mm(s_sc[sb, :] / (d_sc[sb, :] + 1e-16), w2_ref[...], 1, 1)
        outb = outb + bias_ref[...]
        xr = _ln(_elu(xm + outb))
        xc = _transfer(xc_ref, tlw1_ref, tlw2_ref, resc_sc)
        out_ref[...] = _mm(xc + xr, exitw_ref[...], 1, 1)


def kernel(x_centers, x_global, batch_global, tl_w1, tl_w2, tg_w1, tg_w2,
           rc_w1l, rc_w1r, rc_att, rc_w2, rc_bias, exit_w,
           interpret=False):
    n, dim = x_global.shape
    b = x_centers.shape[0]
    nt = -(-n // T)
    npad = nt * T - n
    ids_pad = jnp.pad(batch_global, (0, npad), constant_values=b)
    ids3 = ids_pad.reshape(nt, 1, T)
    los = ids_pad[:: T]
    wides = (ids_pad[T - 1:: T] - los >= W).astype(jnp.int32)
    bias_row = rc_bias.reshape(1, dim)

    full = lambda shape: pl.BlockSpec(shape, lambda i, *_: (0,) * len(shape))
    tile = pl.BlockSpec((T, dim), lambda i, *_: (i, 0))
    idspec = pl.BlockSpec((1, 1, T), lambda i, *_: (i, 0, 0))

    xm_raw, xg_st = pl.pallas_call(
        functools.partial(_body1, nt=nt, b=b, n=n),
        grid_spec=pltpu.PrefetchScalarGridSpec(
            num_scalar_prefetch=2,
            grid=(nt,),
            in_specs=[idspec, tile, full((dim, dim)), full((dim, dim))],
            out_specs=[pl.BlockSpec((b, dim), lambda i, *_: (0, 0)), tile],
            scratch_shapes=[
                pltpu.VMEM((b + W, dim), jnp.float32),
                pltpu.VMEM((T, dim), jnp.float32),
            ],
        ),
        out_shape=[jax.ShapeDtypeStruct((b, dim), jnp.float32),
                   jax.ShapeDtypeStruct((nt * T, dim), jnp.float32)],
        interpret=interpret,
    )(los, wides, ids3, x_global, tg_w1, tg_w2)

    out = pl.pallas_call(
        functools.partial(_body2, nt=nt, b=b),
        grid_spec=pltpu.PrefetchScalarGridSpec(
            num_scalar_prefetch=2,
            grid=(nt,),
            in_specs=[
                idspec, tile, full((b, dim)), full((b, dim)),
                full((dim, dim)), full((dim, dim)),
                full((dim, dim)), full((dim, dim)),
                full((1, dim)), full((dim, dim)), full((1, dim)),
                full((dim, dim)),
            ],
            out_specs=pl.BlockSpec((b, dim), lambda i, *_: (0, 0)),
            scratch_shapes=[
                pltpu.VMEM((b, dim), jnp.float32),
                pltpu.VMEM((b + W, dim), jnp.float32),
                pltpu.VMEM((b + W, 1), jnp.float32),
                pltpu.VMEM((b + W, 1), jnp.float32),
                pltpu.VMEM((b + W, dim), jnp.float32),
            ],
        ),
        out_shape=jax.ShapeDtypeStruct((b, dim), jnp.float32),
        interpret=interpret,
    )(los, wides, ids3, xg_st, xm_raw, x_centers, tl_w1, tl_w2,
      rc_w1l, rc_w1r, rc_att, rc_w2, bias_row, exit_w)
    return out


# speedup vs baseline: 2.3340x; 2.3340x over previous
import functools

import jax
import jax.numpy as jnp
from jax.experimental import pallas as pl
from jax.experimental.pallas import tpu as pltpu

T = 2048
W = 32

NEG_INF = float("-inf")


def _elu(x):
    return jnp.where(x > 0, x, jnp.exp(x) - 1.0)


def _leaky(x):
    return jnp.where(x > 0, x, 0.01 * x)


def _ln(x):
    d = x.shape[-1]
    ones = jnp.ones((d, 1), jnp.float32)
    mu = _mm(x, ones, 1, 0) * (1.0 / d)
    ex2 = _mm(x * x, ones, 1, 0) * (1.0 / d)
    var = ex2 - mu * mu
    return (x - mu) * jax.lax.rsqrt(var + 1e-5)


def _mm(a, b, ca, cb):
    return jax.lax.dot_general(
        a, b, (((ca,), (cb,)), ((), ())), preferred_element_type=jnp.float32)


def _transfer(x_ref, w1_ref, w2_ref, res_sc):
    res_sc[...] = x_ref[...]
    h = _mm(x_ref[...], w1_ref[...], 1, 1)
    h = _ln(_elu(h))
    h = _mm(h, w2_ref[...], 1, 1)
    return _elu(h + res_sc[...])


def _onehot(ids_row, rows):
    seg_iota = jax.lax.broadcasted_iota(jnp.int32, (rows, T), 0)
    ohb = seg_iota == ids_row
    return ohb, ohb.astype(jnp.float32)


def _body1(lo_ref, wide_ref, ids_ref, x_ref, tgw1_ref, tgw2_ref,
           xm_ref, xg_ref, acc_sc, res_sc, *, nt, b, n):
    i = pl.program_id(0)

    @pl.when(i == 0)
    def _init():
        acc_sc[...] = jnp.zeros_like(acc_sc)

    xg = _transfer(x_ref, tgw1_ref, tgw2_ref, res_sc)
    rmask = jax.lax.broadcasted_iota(jnp.int32, (T, 1), 0) < n - i * T
    xg = jnp.where(rmask, xg, 0.0)
    xg_ref[...] = xg
    ids_row = ids_ref[0]
    lo = lo_ref[i]

    @pl.when(wide_ref[i] == 0)
    def _narrow():
        _, oh = _onehot(ids_row - lo, W)
        sl = pl.ds(lo, W)
        acc_sc[sl, :] += _mm(oh, xg, 1, 0)

    @pl.when(wide_ref[i] != 0)
    def _wide():
        _, oh = _onehot(ids_row, b)
        sl = pl.ds(0, b)
        acc_sc[sl, :] += _mm(oh, xg, 1, 0)

    @pl.when(i == nt - 1)
    def _fin():
        xm_ref[...] = acc_sc[pl.ds(0, b), :]


def _accum(ohb, oh, hlw, m_oldw, d_oldw, s_oldw, xg, a1, att):
    hlg = _mm(oh, hlw, 0, 0)
    v = _leaky(a1 + hlg)
    alpha = _leaky(_mm(att, v, 1, 1))
    amT = jnp.where(ohb, alpha, NEG_INF)
    mt = jnp.max(amT, axis=1, keepdims=True)
    m_new = jnp.maximum(m_oldw, mt)
    scale = jnp.where(m_oldw == NEG_INF, 0.0, jnp.exp(m_oldw - m_new))
    m_fin = jnp.where(m_new == NEG_INF, 0.0, m_new)
    mg = _mm(m_fin, oh, 0, 0)
    w = jnp.exp(alpha - mg)
    ohw = oh * w
    ones_col = jnp.ones((T, 1), jnp.float32)
    s_new = s_oldw * scale + _mm(ohw, xg, 1, 0)
    d_new = d_oldw * scale + _mm(ohw, ones_col, 1, 0)
    return m_new, d_new, s_new


def _body2(lo_ref, wide_ref, ids_ref, xg_ref, xm_ref, xc_ref,
           tlw1_ref, tlw2_ref, w1l_ref, w1r_ref, att_ref, w2_ref,
           bias_ref, exitw_ref, out_ref,
           resc_sc, hl_sc, m_sc, d_sc, s_sc, *, nt, b):
    i = pl.program_id(0)

    @pl.when(i == 0)
    def _init():
        hl_sc[pl.ds(0, b), :] = _mm(jax.nn.relu(xm_ref[...]), w1r_ref[...],
                                    1, 1)
        hl_sc[pl.ds(b, W), :] = jnp.zeros((W, hl_sc.shape[1]), jnp.float32)
        m_sc[...] = jnp.full_like(m_sc, NEG_INF)
        d_sc[...] = jnp.zeros_like(d_sc)
        s_sc[...] = jnp.zeros_like(s_sc)

    xg = xg_ref[...]
    a1 = _mm(xg, w1l_ref[...], 1, 1)
    ids_row = ids_ref[0]
    lo = lo_ref[i]
    att = att_ref[...]

    @pl.when(wide_ref[i] == 0)
    def _narrow():
        ohb, oh = _onehot(ids_row - lo, W)
        sl = pl.ds(lo, W)
        m_new, d_new, s_new = _accum(
            ohb, oh, hl_sc[sl, :], m_sc[sl, :], d_sc[sl, :], s_sc[sl, :],
            xg, a1, att)
        m_sc[sl, :] = m_new
        d_sc[sl, :] = d_new
        s_sc[sl, :] = s_new

    @pl.when(wide_ref[i] != 0)
    def _wide():
        ohb, oh = _onehot(ids_row, b)
        sl = pl.ds(0, b)
        m_new, d_new, s_new = _accum(
            ohb, oh, hl_sc[sl, :], m_sc[sl, :], d_sc[sl, :], s_sc[sl, :],
            xg, a1, att)
        m_sc[sl, :] = m_new
        d_sc[sl, :] = d_new
        s_sc[sl, :] = s_new

    @pl.when(i == nt - 1)
    def _fin():
        xm = jax.nn.relu(xm_ref[...])
        sb = pl.ds(0, b)
        outb = _mm(s_sc[sb, :] / (d_sc[sb, :] + 1e-16), w2_ref[...], 1, 1)
        outb = outb + bias_ref[...]
        xr = _ln(_elu(xm + outb))
        xc = _transfer(xc_ref, tlw1_ref, tlw2_ref, resc_sc)
        out_ref[...] = _mm(xc + xr, exitw_ref[...], 1, 1)


def kernel(x_centers, x_global, batch_global, tl_w1, tl_w2, tg_w1, tg_w2,
           rc_w1l, rc_w1r, rc_att, rc_w2, rc_bias, exit_w,
           interpret=False):
    n, dim = x_global.shape
    b = x_centers.shape[0]
    nt = -(-n // T)
    npad = nt * T - n
    ids_pad = jnp.pad(batch_global, (0, npad), constant_values=b)
    ids3 = ids_pad.reshape(nt, 1, T)
    los = ids_pad[:: T]
    wides = (ids_pad[T - 1:: T] - los >= W).astype(jnp.int32)
    bias_row = rc_bias.reshape(1, dim)

    full = lambda shape: pl.BlockSpec(shape, lambda i, *_: (0,) * len(shape))
    tile = pl.BlockSpec((T, dim), lambda i, *_: (i, 0))
    idspec = pl.BlockSpec((1, 1, T), lambda i, *_: (i, 0, 0))

    xm_raw, xg_st = pl.pallas_call(
        functools.partial(_body1, nt=nt, b=b, n=n),
        grid_spec=pltpu.PrefetchScalarGridSpec(
            num_scalar_prefetch=2,
            grid=(nt,),
            in_specs=[idspec, tile, full((dim, dim)), full((dim, dim))],
            out_specs=[pl.BlockSpec((b, dim), lambda i, *_: (0, 0)), tile],
            scratch_shapes=[
                pltpu.VMEM((b + W, dim), jnp.float32),
                pltpu.VMEM((T, dim), jnp.float32),
            ],
        ),
        out_shape=[jax.ShapeDtypeStruct((b, dim), jnp.float32),
                   jax.ShapeDtypeStruct((nt * T, dim), jnp.float32)],
        interpret=interpret,
    )(los, wides, ids3, x_global, tg_w1, tg_w2)

    out = pl.pallas_call(
        functools.partial(_body2, nt=nt, b=b),
        grid_spec=pltpu.PrefetchScalarGridSpec(
            num_scalar_prefetch=2,
            grid=(nt,),
            in_specs=[
                idspec, tile, full((b, dim)), full((b, dim)),
                full((dim, dim)), full((dim, dim)),
                full((dim, dim)), full((dim, dim)),
                full((1, dim)), full((dim, dim)), full((1, dim)),
                full((dim, dim)),
            ],
            out_specs=pl.BlockSpec((b, dim), lambda i, *_: (0, 0)),
            scratch_shapes=[
                pltpu.VMEM((b, dim), jnp.float32),
                pltpu.VMEM((b + W, dim), jnp.float32),
                pltpu.VMEM((b + W, 1), jnp.float32),
                pltpu.VMEM((b + W, 1), jnp.float32),
                pltpu.VMEM((b + W, dim), jnp.float32),
            ],
        ),
        out_shape=jax.ShapeDtypeStruct((b, dim), jnp.float32),
        interpret=interpret,
    )(los, wides, ids3, xg_st, xm_raw, x_centers, tl_w1, tl_w2,
      rc_w1l, rc_w1r, rc_att, rc_w2, bias_row, exit_w)
    return out
